# baseline (speedup 1.0000x reference)
import functools

import jax
import jax.numpy as jnp
from jax import lax
from jax.experimental import pallas as pl
from jax.experimental.pallas import tpu as pltpu
from jax.experimental.pallas import tpu_sc as plsc

NC = 2
NS = 16
NW = NC * NS
LANES = 16
DCHUNK = 128
CHUNK = 128


def _sc_mesh():
    return plsc.VectorSubcoreMesh(
        core_axis_name="c", subcore_axis_name="s", num_cores=NC, num_subcores=NS
    )


def _make_deg_kernel(n_pad, cpt):
    @functools.partial(
        pl.kernel,
        out_type=jax.ShapeDtypeStruct((NC, NS, n_pad), jnp.float32),
        mesh=_sc_mesh(),
        compiler_params=pltpu.CompilerParams(needs_layout_passes=False),
        scratch_types=[
            pltpu.VMEM((cpt, DCHUNK), jnp.int32),
            pltpu.VMEM((n_pad,), jnp.float32),
        ],
    )
    def deg_kernel(dst_hbm, degp_hbm, idx_v, cnt_v):
        cid = lax.axis_index("c")
        sid = lax.axis_index("s")
        zeros = jnp.zeros((LANES,), jnp.float32)

        @pl.loop(0, n_pad // LANES)
        def _(i):
            cnt_v[pl.ds(i * LANES, LANES)] = zeros

        pltpu.sync_copy(dst_hbm.at[cid, sid], idx_v)
        ones = jnp.ones((LANES,), jnp.float32)

        @pl.loop(0, cpt)
        def _(c):
            for k in range(DCHUNK // LANES):
                idx = idx_v[c, pl.ds(k * LANES, LANES)]
                plsc.addupdate_scatter(cnt_v, [idx], ones)

        pltpu.sync_copy(cnt_v, degp_hbm.at[cid, sid])

    return deg_kernel


def _make_scatter_kernel(n, d, n_pad, cpt):
    z_rows_per_tile = n_pad // NS
    ept = cpt * CHUNK

    @functools.partial(
        pl.kernel,
        out_type=jax.ShapeDtypeStruct((NC, n_pad, d), jnp.float32),
        mesh=_sc_mesh(),
        scratch_types=[
            pltpu.VMEM((ept,), jnp.int32),
            pltpu.VMEM((2, CHUNK), jnp.int32),
            pltpu.VMEM((CHUNK, d), jnp.float32),
            pltpu.VMEM((CHUNK, d), jnp.float32),
            pltpu.VMEM_SHARED((n_pad, d), jnp.float32),
            pltpu.SemaphoreType.DMA,
            pltpu.SemaphoreType.DMA,
            pltpu.SemaphoreType.DMA,
            pltpu.SemaphoreType.DMA,
        ],
    )
    def scatter_kernel(y_hbm, src_hbm, dst_hbm, z_hbm,
                       src_v, dstr, buf0, buf1, z_sh,
                       gsem0, gsem1, dsem0, dsem1):
        cid = lax.axis_index("c")
        sid = lax.axis_index("s")

        pltpu.sync_copy(src_hbm.at[cid, sid], src_v)

        zeros = jnp.zeros((LANES,), jnp.float32)

        @pl.loop(0, CHUNK)
        def _(r):
            for k in range(d // LANES):
                buf0[r, pl.ds(k * LANES, LANES)] = zeros

        base = sid * z_rows_per_tile
        for k in range(z_rows_per_tile // CHUNK):
            pltpu.sync_copy(buf0, z_sh.at[pl.ds(base + k * CHUNK, CHUNK)])
        plsc.subcore_barrier()

        bufs = (buf0, buf1)
        gsems = (gsem0, gsem1)
        dsems = (dsem0, dsem1)
        for s in (0, 1):
            pltpu.async_copy(dst_hbm.at[cid, sid, s], dstr.at[s], dsems[s])
            pltpu.async_copy(y_hbm.at[src_v.at[pl.ds(s * CHUNK, CHUNK)]],
                             bufs[s], gsems[s])

        @pl.loop(0, cpt, step=2)
        def _(jo):
            for b in range(2):
                j = jo + b
                pltpu.make_async_copy(dst_hbm.at[cid, sid, 0], dstr.at[b],
                                      dsems[b]).wait()
                pltpu.make_async_copy(y_hbm.at[src_v.at[pl.ds(0, CHUNK)]],
                                      bufs[b], gsems[b]).wait()
                pltpu.sync_copy(bufs[b], z_sh.at[dstr.at[b]], add=True)

                @pl.when(j + 2 < cpt)
                def _():
                    pltpu.async_copy(dst_hbm.at[cid, sid, j + 2], dstr.at[b],
                                     dsems[b])
                    pltpu.async_copy(
                        y_hbm.at[src_v.at[pl.ds((j + 2) * CHUNK, CHUNK)]],
                        bufs[b], gsems[b])

        plsc.subcore_barrier()
        pltpu.sync_copy(z_sh.at[pl.ds(base, z_rows_per_tile)],
                        z_hbm.at[cid, pl.ds(base, z_rows_per_tile)])

    return scatter_kernel


def _make_mm_kernel(n, d, bm):
    def body(x_ref, w_ref, xw_ref):
        xw_ref[...] = jnp.dot(x_ref[...], w_ref[...],
                              preferred_element_type=jnp.float32)

    grid = pl.cdiv(n, bm)
    return pl.pallas_call(
        body,
        grid=(grid,),
        in_specs=[
            pl.BlockSpec((bm, d), lambda i: (i, 0)),
            pl.BlockSpec((d, d), lambda i: (0, 0)),
        ],
        out_specs=pl.BlockSpec((bm, d), lambda i: (i, 0)),
        out_shape=jax.ShapeDtypeStruct((n, d), jnp.float32),
    )


def _make_scale_kernel(n, d, bm):
    def body(xw_ref, degp_ref, y_ref, dinv_ref):
        cnt = jnp.sum(degp_ref[...], axis=0)
        dinv = lax.rsqrt(cnt + 1.0)
        y_ref[...] = xw_ref[...] * dinv[:, None]
        dinv_ref[...] = dinv[:, None]

    grid = pl.cdiv(n, bm)
    return pl.pallas_call(
        body,
        grid=(grid,),
        in_specs=[
            pl.BlockSpec((bm, d), lambda i: (i, 0)),
            pl.BlockSpec((NW, bm), lambda i: (0, i)),
        ],
        out_specs=[
            pl.BlockSpec((bm, d), lambda i: (i, 0)),
            pl.BlockSpec((bm, 1), lambda i: (i, 0)),
        ],
        out_shape=[
            jax.ShapeDtypeStruct((n, d), jnp.float32),
            jax.ShapeDtypeStruct((n, 1), jnp.float32),
        ],
    )


def _make_head_kernel(n, d, bm):
    def body(z_ref, y_ref, dinv_ref, bc_ref, w1_ref, b1_ref, w2_ref, b2_ref,
             o_ref):
        zsum = z_ref[0] + z_ref[1] + y_ref[...]
        h = jnp.maximum(zsum * dinv_ref[...] + bc_ref[...], 0.0)
        h = jnp.maximum(
            jnp.dot(h, w1_ref[...], preferred_element_type=jnp.float32)
            + b1_ref[...], 0.0)
        h = jnp.maximum(
            jnp.dot(h, w2_ref[...], preferred_element_type=jnp.float32)
            + b2_ref[...], 0.0)
        o_ref[...] = jax.nn.sigmoid(h)

    grid = pl.cdiv(n, bm)
    return pl.pallas_call(
        body,
        grid=(grid,),
        in_specs=[
            pl.BlockSpec((NC, bm, d), lambda i: (0, i, 0)),
            pl.BlockSpec((bm, d), lambda i: (i, 0)),
            pl.BlockSpec((bm, 1), lambda i: (i, 0)),
            pl.BlockSpec((1, d), lambda i: (0, 0)),
            pl.BlockSpec((d, d), lambda i: (0, 0)),
            pl.BlockSpec((1, d), lambda i: (0, 0)),
            pl.BlockSpec((d, 1), lambda i: (0, 0)),
            pl.BlockSpec((1, 1), lambda i: (0, 0)),
        ],
        out_specs=pl.BlockSpec((bm, 1), lambda i: (i, 0)),
        out_shape=jax.ShapeDtypeStruct((n, 1), jnp.float32),
    )


def kernel(x, edge_index, W_conv, b_conv, W_lin1, b_lin1, W_lin2, b_lin2):
    n, d = x.shape
    e = edge_index.shape[1]

    cpt_deg = pl.cdiv(e, NW * DCHUNK)
    cpt_deg = cpt_deg + (cpt_deg % 2)
    e_pad = NW * cpt_deg * DCHUNK
    cpt_sc = (e_pad // NW) // CHUNK
    n_pad = ((n + NS * CHUNK - 1) // (NS * CHUNK)) * (NS * CHUNK)
    if n_pad == n:
        n_pad += NS * CHUNK

    src = edge_index[0]
    dst = edge_index[1]
    pad = e_pad - e
    pad_ids = jnp.arange(pad, dtype=jnp.int32)
    srcp = jnp.concatenate([src, pad_ids % n])
    dstp = jnp.concatenate([dst, n + pad_ids % (n_pad - n)])

    degp = _make_deg_kernel(n_pad, cpt_deg)(
        dstp.reshape(NC, NS, cpt_deg, DCHUNK))
    xw = _make_mm_kernel(n, d, 256)(x, W_conv)
    y, dinv = _make_scale_kernel(n, d, 256)(xw, degp.reshape(NW, n_pad))
    z = _make_scatter_kernel(n, d, n_pad, cpt_sc)(
        y, srcp.reshape(NC, NS, cpt_sc * CHUNK),
        dstp.reshape(NC, NS, cpt_sc, CHUNK))
    out = _make_head_kernel(n, d, 256)(
        z, y, dinv, b_conv.reshape(1, d), W_lin1, b_lin1.reshape(1, d),
        W_lin2, b_lin2.reshape(1, 1))
    return out


# speedup vs baseline: 1.3210x; 1.3210x over previous
import functools

import numpy as np

import jax
import jax.numpy as jnp
from jax import lax
from jax.experimental import pallas as pl
from jax.experimental.pallas import tpu as pltpu
from jax.experimental.pallas import tpu_sc as plsc

NC = 2
NS = 16
NW = NC * NS
LANES = 16
DCHUNK = 128
CHUNK = 128


def _sc_mesh():
    return plsc.VectorSubcoreMesh(
        core_axis_name="c", subcore_axis_name="s", num_cores=NC, num_subcores=NS
    )


def _make_deg_kernel(n_pad, cpt):
    @functools.partial(
        pl.kernel,
        out_type=jax.ShapeDtypeStruct((NC, NS, n_pad), jnp.float32),
        mesh=_sc_mesh(),
        compiler_params=pltpu.CompilerParams(needs_layout_passes=False),
        scratch_types=[
            pltpu.VMEM((cpt, DCHUNK), jnp.int32),
            pltpu.VMEM((n_pad,), jnp.float32),
        ],
    )
    def deg_kernel(dst_hbm, degp_hbm, idx_v, cnt_v):
        cid = lax.axis_index("c")
        sid = lax.axis_index("s")
        zeros = jnp.zeros((LANES,), jnp.float32)

        @pl.loop(0, n_pad // LANES)
        def _(i):
            cnt_v[pl.ds(i * LANES, LANES)] = zeros

        pltpu.sync_copy(dst_hbm.at[cid, sid], idx_v)
        ones = jnp.ones((LANES,), jnp.float32)

        @pl.loop(0, cpt)
        def _(c):
            for k in range(DCHUNK // LANES):
                idx = idx_v[c, pl.ds(k * LANES, LANES)]
                plsc.addupdate_scatter(cnt_v, [idx], ones)

        pltpu.sync_copy(cnt_v, degp_hbm.at[cid, sid])

    return deg_kernel


def _make_scatter_kernel(n, d, n_pad, cpt):
    z_rows_per_tile = n_pad // NS
    ept = cpt * CHUNK

    @functools.partial(
        pl.kernel,
        out_type=jax.ShapeDtypeStruct((NC, n_pad, d), jnp.float32),
        mesh=_sc_mesh(),
        scratch_types=[
            pltpu.VMEM((ept,), jnp.int32),
            pltpu.VMEM((2, CHUNK), jnp.int32),
            pltpu.VMEM((CHUNK, d), jnp.float32),
            pltpu.VMEM((CHUNK, d), jnp.float32),
            pltpu.VMEM_SHARED((n_pad, d), jnp.float32),
            pltpu.SemaphoreType.DMA,
            pltpu.SemaphoreType.DMA,
            pltpu.SemaphoreType.DMA,
            pltpu.SemaphoreType.DMA,
        ],
    )
    def scatter_kernel(y_hbm, src_hbm, dst_hbm, z_hbm,
                       src_v, dstr, buf0, buf1, z_sh,
                       gsem0, gsem1, dsem0, dsem1):
        cid = lax.axis_index("c")
        sid = lax.axis_index("s")

        pltpu.sync_copy(src_hbm.at[cid, sid], src_v)

        zeros = jnp.zeros((LANES,), jnp.float32)

        @pl.loop(0, CHUNK)
        def _(r):
            for k in range(d // LANES):
                buf0[r, pl.ds(k * LANES, LANES)] = zeros

        base = sid * z_rows_per_tile
        for k in range(z_rows_per_tile // CHUNK):
            pltpu.sync_copy(buf0, z_sh.at[pl.ds(base + k * CHUNK, CHUNK)])
        plsc.subcore_barrier()

        bufs = (buf0, buf1)
        gsems = (gsem0, gsem1)
        dsems = (dsem0, dsem1)
        for s in (0, 1):
            pltpu.async_copy(dst_hbm.at[cid, sid, s], dstr.at[s], dsems[s])
            pltpu.async_copy(y_hbm.at[src_v.at[pl.ds(s * CHUNK, CHUNK)]],
                             bufs[s], gsems[s])

        @pl.loop(0, cpt, step=2)
        def _(jo):
            for b in range(2):
                j = jo + b
                pltpu.make_async_copy(dst_hbm.at[cid, sid, 0], dstr.at[b],
                                      dsems[b]).wait()
                pltpu.make_async_copy(y_hbm.at[src_v.at[pl.ds(0, CHUNK)]],
                                      bufs[b], gsems[b]).wait()
                pltpu.sync_copy(bufs[b], z_sh.at[dstr.at[b]], add=True)

                @pl.when(j + 2 < cpt)
                def _():
                    pltpu.async_copy(dst_hbm.at[cid, sid, j + 2], dstr.at[b],
                                     dsems[b])
                    pltpu.async_copy(
                        y_hbm.at[src_v.at[pl.ds((j + 2) * CHUNK, CHUNK)]],
                        bufs[b], gsems[b])

        plsc.subcore_barrier()
        pltpu.sync_copy(z_sh.at[pl.ds(base, z_rows_per_tile)],
                        z_hbm.at[cid, pl.ds(base, z_rows_per_tile)])

    return scatter_kernel


def _make_prep_kernel(n, d, n_pad):
    def body(x_ref, w_ref, degp_ref, y_ref, dinv_ref):
        cnt = jnp.sum(degp_ref[...], axis=0)[:n]
        dinv = lax.rsqrt(cnt + 1.0)
        xw = jnp.dot(x_ref[...], w_ref[...], preferred_element_type=jnp.float32)
        y_ref[...] = xw * dinv[:, None]
        dinv_ref[...] = dinv[:, None]

    return pl.pallas_call(
        body,
        grid=(1,),
        in_specs=[
            pl.BlockSpec((n, d), lambda i: (0, 0)),
            pl.BlockSpec((d, d), lambda i: (0, 0)),
            pl.BlockSpec((NW, n_pad), lambda i: (0, 0)),
        ],
        out_specs=[
            pl.BlockSpec((n, d), lambda i: (0, 0)),
            pl.BlockSpec((n, 1), lambda i: (0, 0)),
        ],
        out_shape=[
            jax.ShapeDtypeStruct((n, d), jnp.float32),
            jax.ShapeDtypeStruct((n, 1), jnp.float32),
        ],
    )


def _make_head_kernel(n, d):
    def body(z_ref, y_ref, dinv_ref, bc_ref, w1_ref, b1_ref, w2_ref, b2_ref,
             o_ref):
        zsum = z_ref[0] + z_ref[1] + y_ref[...]
        h = jnp.maximum(zsum * dinv_ref[...] + bc_ref[...], 0.0)
        h = jnp.maximum(
            jnp.dot(h, w1_ref[...], preferred_element_type=jnp.float32)
            + b1_ref[...], 0.0)
        h = jnp.maximum(
            jnp.dot(h, w2_ref[...], preferred_element_type=jnp.float32)
            + b2_ref[...], 0.0)
        o_ref[...] = jax.nn.sigmoid(h)

    return pl.pallas_call(
        body,
        grid=(1,),
        in_specs=[
            pl.BlockSpec((NC, n, d), lambda i: (0, 0, 0)),
            pl.BlockSpec((n, d), lambda i: (0, 0)),
            pl.BlockSpec((n, 1), lambda i: (0, 0)),
            pl.BlockSpec((1, d), lambda i: (0, 0)),
            pl.BlockSpec((d, d), lambda i: (0, 0)),
            pl.BlockSpec((1, d), lambda i: (0, 0)),
            pl.BlockSpec((d, 1), lambda i: (0, 0)),
            pl.BlockSpec((1, 1), lambda i: (0, 0)),
        ],
        out_specs=pl.BlockSpec((n, 1), lambda i: (0, 0)),
        out_shape=jax.ShapeDtypeStruct((n, 1), jnp.float32),
    )


def kernel(x, edge_index, W_conv, b_conv, W_lin1, b_lin1, W_lin2, b_lin2):
    n, d = x.shape
    e = edge_index.shape[1]

    cpt_deg = pl.cdiv(e, NW * DCHUNK)
    cpt_deg = cpt_deg + (cpt_deg % 2)
    e_pad = NW * cpt_deg * DCHUNK
    cpt_sc = (e_pad // NW) // CHUNK
    n_pad = ((n + NS * CHUNK - 1) // (NS * CHUNK)) * (NS * CHUNK)
    if n_pad == n:
        n_pad += NS * CHUNK

    src = edge_index[0]
    dst = edge_index[1]
    pad = e_pad - e
    pad_ids = np.arange(pad, dtype=np.int32)
    srcp = jnp.concatenate([src, jnp.asarray(pad_ids % n)])
    dstp = jnp.concatenate([dst, jnp.asarray(n + pad_ids % (n_pad - n))])

    degp = _make_deg_kernel(n_pad, cpt_deg)(
        dstp.reshape(NC, NS, cpt_deg, DCHUNK))
    y, dinv = _make_prep_kernel(n, d, n_pad)(
        x, W_conv, degp.reshape(NW, n_pad))
    z = _make_scatter_kernel(n, d, n_pad, cpt_sc)(
        y, srcp.reshape(NC, NS, cpt_sc * CHUNK),
        dstp.reshape(NC, NS, cpt_sc, CHUNK))
    out = _make_head_kernel(n, d)(
        z, y, dinv, b_conv.reshape(1, d), W_lin1, b_lin1.reshape(1, d),
        W_lin2, b_lin2.reshape(1, 1))
    return out
